# baseline (speedup 1.0000x reference)
import jax
import jax.numpy as jnp
from jax.experimental import pallas as pl
from jax.experimental.pallas import tpu as pltpu

_CH = 1024
_NS = 4


def _make_body(B, T, C):
    NP = T // _CH
    W = NP * B

    def _body(xf_ref, pe_ref, o_ref, xb, peb, ob, sx, sp, so):
        def xrow(w):
            p, b = divmod(w, B)
            return b * T + p * _CH

        def x_copy(w):
            return pltpu.make_async_copy(
                xf_ref.at[pl.ds(xrow(w), _CH)], xb.at[w % _NS], sx.at[w % _NS]
            )

        def pe_copy(p):
            return pltpu.make_async_copy(
                pe_ref.at[pl.ds(p * _CH, _CH)], peb.at[p % 2], sp.at[p % 2]
            )

        def o_copy(w):
            return pltpu.make_async_copy(
                ob.at[w % _NS], o_ref.at[pl.ds(xrow(w), _CH)], so.at[w % _NS]
            )

        pe_copy(0).start()
        for w in range(min(_NS, W)):
            x_copy(w).start()
        if NP > 1:
            pe_copy(1).start()
        for w in range(W):
            p, b = divmod(w, B)
            if b == 0:
                pe_copy(p).wait()
            x_copy(w).wait()
            if w >= _NS:
                o_copy(w - _NS).wait()
            ob[w % _NS] = xb[w % _NS] + peb[p % 2]
            o_copy(w).start()
            if w + _NS < W:
                x_copy(w + _NS).start()
            if b == B - 1 and p + 2 < NP:
                pe_copy(p + 2).start()
        for w in range(max(0, W - _NS), W):
            o_copy(w).wait()

    return _body


def kernel(x, pos_embed):
    B, T, C = x.shape
    pe = pos_embed[:T]
    xf = x.reshape(B * T, C)
    out = pl.pallas_call(
        _make_body(B, T, C),
        in_specs=[
            pl.BlockSpec(memory_space=pl.ANY),
            pl.BlockSpec(memory_space=pl.ANY),
        ],
        out_specs=pl.BlockSpec(memory_space=pl.ANY),
        out_shape=jax.ShapeDtypeStruct((B * T, C), x.dtype),
        scratch_shapes=[
            pltpu.VMEM((_NS, _CH, C), x.dtype),
            pltpu.VMEM((2, _CH, C), x.dtype),
            pltpu.VMEM((_NS, _CH, C), x.dtype),
            pltpu.SemaphoreType.DMA((_NS,)),
            pltpu.SemaphoreType.DMA((2,)),
            pltpu.SemaphoreType.DMA((_NS,)),
        ],
    )(xf, pe)
    return out.reshape(B, T, C)
